# baseline (speedup 1.0000x reference)
import jax
import jax.numpy as jnp
from jax.experimental import pallas as pl
from jax.experimental.pallas import tpu as pltpu

_SPLIT = 4
_DN = (((1,), (1,)), ((), ()))

_BLOCKS = (
    (1, 0, 256), (1, 256, 512), (1, 768, 256),
    (2, 0, 512), (2, 512, 256), (2, 768, 256),
)
_MAXROWS = 512


def _row_splits(nrows):
    per = (nrows // _SPLIT + 7) // 8 * 8
    offs, o = [], 0
    while o < nrows:
        sz = min(per, nrows - o)
        offs.append((o, sz))
        o += sz
    return offs


def _mlp_kernel(x_hbm, w1_hbm, b1_hbm, w2_hbm, b2_hbm, w3_hbm, b3_hbm,
                o_ref, xv, wbuf, h1, oacc, w3v, b1v, b2v, b3v,
                sem_w, sem_x, sem_w3, sem_b):
    n = len(_BLOCKS)

    def w_copies(t):
        layer, r0, rows = _BLOCKS[t]
        w_hbm = w1_hbm if layer == 1 else w2_hbm
        return [pltpu.make_async_copy(
                    w_hbm.at[pl.ds(r0 + o, sz), :],
                    wbuf.at[t % 2, pl.ds(o, sz), :],
                    sem_w.at[t % 2])
                for o, sz in _row_splits(rows)]

    def w_wait(t):
        rows = _BLOCKS[t][2]
        pltpu.make_async_copy(
            w1_hbm.at[pl.ds(0, rows), :],
            wbuf.at[t % 2, pl.ds(0, rows), :],
            sem_w.at[t % 2]).wait()

    cp_x = [pltpu.make_async_copy(x_hbm.at[pl.ds(i * 128, 128), :],
                                  xv.at[pl.ds(i * 128, 128), :], sem_x.at[i])
            for i in range(2)]
    cp_b1 = pltpu.make_async_copy(b1_hbm, b1v, sem_b.at[0])
    cp_b2 = pltpu.make_async_copy(b2_hbm, b2v, sem_b.at[1])
    cp_b3 = pltpu.make_async_copy(b3_hbm, b3v, sem_b.at[2])
    cp_w3 = pltpu.make_async_copy(w3_hbm, w3v, sem_w3)

    for c in cp_x:
        c.start()
    cp_b1.start()
    cp_b2.start()
    for c in w_copies(0):
        c.start()

    for c in cp_x:
        c.wait()
    cp_b1.wait()
    cp_b2.wait()

    for t in range(n):
        w_wait(t)
        if t + 1 < n:
            for c in w_copies(t + 1):
                c.start()
        if t == 0:
            cp_w3.start()
            cp_b3.start()
        layer, r0, rows = _BLOCKS[t]
        wblk = wbuf[t % 2, pl.ds(0, rows), :]
        if layer == 1:
            h = jax.lax.dot_general(xv[...], wblk, _DN,
                                    preferred_element_type=jnp.float32)
            h1[:, pl.ds(r0, rows)] = jnp.maximum(
                h + b1v[:, pl.ds(r0, rows)], 0.0)
        else:
            h = jax.lax.dot_general(h1[...], wblk, _DN,
                                    preferred_element_type=jnp.float32)
            h2blk = jnp.maximum(h + b2v[:, pl.ds(r0, rows)], 0.0)
            if r0 == 0:
                cp_w3.wait()
                cp_b3.wait()
            part = jax.lax.dot_general(h2blk, w3v[:, pl.ds(r0, rows)], _DN,
                                       preferred_element_type=jnp.float32)
            if r0 == 0:
                oacc[...] = part + b3v[...]
            else:
                oacc[...] += part

    o_ref[...] = oacc[...]


def kernel(x, W1, b1, W2, b2, W3, b3, t):
    del t
    B, D_IN = x.shape
    D_H = W1.shape[0]
    D_OUT = W3.shape[0]
    return pl.pallas_call(
        _mlp_kernel,
        in_specs=[pl.BlockSpec(memory_space=pl.ANY)] * 7,
        out_specs=pl.BlockSpec((B, D_OUT), lambda: (0, 0)),
        out_shape=jax.ShapeDtypeStruct((B, D_OUT), jnp.float32),
        scratch_shapes=[
            pltpu.VMEM((B, D_IN), jnp.float32),
            pltpu.VMEM((2, _MAXROWS, D_IN), jnp.float32),
            pltpu.VMEM((B, D_H), jnp.float32),
            pltpu.VMEM((B, D_OUT), jnp.float32),
            pltpu.VMEM((D_OUT, D_H), jnp.float32),
            pltpu.VMEM((1, D_H), jnp.float32),
            pltpu.VMEM((1, D_H), jnp.float32),
            pltpu.VMEM((1, D_OUT), jnp.float32),
            pltpu.SemaphoreType.DMA((2,)),
            pltpu.SemaphoreType.DMA((2,)),
            pltpu.SemaphoreType.DMA,
            pltpu.SemaphoreType.DMA((3,)),
        ],
    )(x, W1, b1.reshape(1, -1), W2, b2.reshape(1, -1), W3, b3.reshape(1, -1))
